# baseline (speedup 1.0000x reference)
import functools

import jax
import jax.numpy as jnp
from jax import lax
from jax.experimental import pallas as pl
from jax.experimental.pallas import tpu as pltpu
from jax.experimental.pallas import tpu_sc as plsc

B, P, C, D = 64, 1024, 14, 768
K = 16
L = 16
NW = 32
NCHUNK = P // L
SPLIT = 32
NSC = B - SPLIT
NGRP = P // L


def _pb_body(lbl_ref, proto_ref, out_ref):
    p = proto_ref[0, 0, :]
    pn = p / (jnp.sqrt(jnp.sum(p * p)) + 1e-8)
    out_ref[0, 0, :] = pn.astype(jnp.bfloat16).astype(jnp.float32)


def _build_pbs(prototypes, labels):
    grid_spec = pltpu.PrefetchScalarGridSpec(
        num_scalar_prefetch=1,
        grid=(B,),
        in_specs=[
            pl.BlockSpec((1, 1, D), lambda b, lbl: (b * C + lbl[b], 0, 0)),
        ],
        out_specs=pl.BlockSpec((1, 1, D), lambda b, lbl: (b, 0, 0)),
    )
    pbs3 = pl.pallas_call(
        _pb_body,
        grid_spec=grid_spec,
        out_shape=jax.ShapeDtypeStruct((B, 1, D), jnp.float32),
    )(labels, prototypes.reshape(B * C, 1, D))
    return pbs3


def _sim_body(lf_ref, pb_ref, out_ref):
    pb = pb_ref[0, 0, :]
    x = lf_ref[0]
    nrm = jnp.sqrt(jnp.sum(x * x, axis=1)) + 1e-8
    xb = (x / nrm[:, None]).astype(jnp.bfloat16).astype(jnp.float32)
    out_ref[0, 0, :] = jnp.sum(xb * pb[None, :], axis=1)


def _similarity(local_features, pbs):
    sim3 = pl.pallas_call(
        _sim_body,
        grid=(SPLIT,),
        in_specs=[
            pl.BlockSpec((1, P, D), lambda b: (b, 0, 0)),
            pl.BlockSpec((1, 1, D), lambda b: (b, 0, 0)),
        ],
        out_specs=pl.BlockSpec((1, 1, P), lambda b: (b, 0, 0)),
        out_shape=jax.ShapeDtypeStruct((SPLIT, 1, P), jnp.float32),
    )(local_features, pbs)
    return sim3.reshape(SPLIT, P)


def _merge_top(run_k, run_v, cand_k, cand_v):
    rb_k = lax.rev(cand_k, (0,))
    rb_v = lax.rev(cand_v, (0,))
    keep = run_k >= rb_k
    mk = jnp.where(keep, run_k, rb_k)
    mv = jnp.where(keep, run_v, rb_v)
    return plsc.sort_key_val(mk, mv)


def _merge_bot(run_k, run_v, cand_k, cand_v):
    rb_k = lax.rev(cand_k, (0,))
    rb_v = lax.rev(cand_v, (0,))
    keep = run_k <= rb_k
    mk = jnp.where(keep, run_k, rb_k)
    mv = jnp.where(keep, run_v, rb_v)
    return plsc.sort_key_val(mk, mv)


_SEL_INIT = lambda: (
    jnp.full((L,), -2.0, jnp.float32), jnp.zeros((L,), jnp.int32),
    jnp.full((L,), 2.0, jnp.float32), jnp.zeros((L,), jnp.int32),
)


def _emit_selection(tv, bv, row0, lf_hbm, tf_slot, bf_slot, ti_slot, bi_slot,
                    ti_v, bi_v, gt_v, gb_v, tr_v, br_v, sem0, sem1):
    top_idx = lax.rev(tv, (0,))
    bot_idx = bv
    ti_v[...] = top_idx
    bi_v[...] = bot_idx
    gt_v[...] = top_idx + row0
    gb_v[...] = bot_idx + row0
    cp_t = pltpu.async_copy(lf_hbm.at[gt_v], tr_v, sem0)
    cp_b = pltpu.async_copy(lf_hbm.at[gb_v], br_v, sem1)
    pltpu.sync_copy(ti_v, ti_slot)
    pltpu.sync_copy(bi_v, bi_slot)
    cp_t.wait()
    cp_b.wait()
    pltpu.sync_copy(tr_v, tf_slot)
    pltpu.sync_copy(br_v, bf_slot)


def _select_body(sim_hbm, lf_hbm, tf_hbm, bf_hbm, ti_hbm, bi_hbm,
                 sim_v, ti_v, bi_v, gt_v, gb_v, tr_v, br_v, sem0, sem1):
    wid = lax.axis_index("s") * 2 + lax.axis_index("c")
    base_iota = lax.iota(jnp.int32, L)

    @pl.when(wid < SPLIT)
    def _():
        b = wid
        pltpu.sync_copy(sim_hbm.at[b], sim_v)

        def chunk_step(c, carry):
            tk, tv, bk, bv = carry
            chunk = sim_v[pl.ds(c * L, L)]
            cidx = base_iota + c * L
            sk, sv = plsc.sort_key_val(chunk, cidx)
            tk, tv = _merge_top(tk, tv, sk, sv)
            bk, bv = _merge_bot(bk, bv, sk, sv)
            return tk, tv, bk, bv

        _, tv, _, bv = lax.fori_loop(0, NCHUNK, chunk_step, _SEL_INIT())
        _emit_selection(tv, bv, b * P, lf_hbm,
                        tf_hbm.at[b], bf_hbm.at[b], ti_hbm.at[b], bi_hbm.at[b],
                        ti_v, bi_v, gt_v, gb_v, tr_v, br_v, sem0, sem1)


def _select(sim, lf_flat):
    mesh = plsc.VectorSubcoreMesh(core_axis_name="c", subcore_axis_name="s")
    out_type = (
        jax.ShapeDtypeStruct((SPLIT, K, D), jnp.float32),
        jax.ShapeDtypeStruct((SPLIT, K, D), jnp.float32),
        jax.ShapeDtypeStruct((SPLIT, K), jnp.int32),
        jax.ShapeDtypeStruct((SPLIT, K), jnp.int32),
    )
    scratch = [
        pltpu.VMEM((P,), jnp.float32),
        pltpu.VMEM((K,), jnp.int32),
        pltpu.VMEM((K,), jnp.int32),
        pltpu.VMEM((K,), jnp.int32),
        pltpu.VMEM((K,), jnp.int32),
        pltpu.VMEM((K, D), jnp.float32),
        pltpu.VMEM((K, D), jnp.float32),
        pltpu.SemaphoreType.DMA,
        pltpu.SemaphoreType.DMA,
    ]
    run = pl.kernel(_select_body, out_type=out_type, mesh=mesh,
                    scratch_types=scratch,
                    compiler_params=pltpu.CompilerParams(
                        needs_layout_passes=False))
    return run(sim, lf_flat)


def _round_bf16(x):
    i = lax.bitcast_convert_type(x, jnp.int32)
    r = (i + jnp.int32(0x7FFF) + ((i >> 16) & 1)) & jnp.int32(-65536)
    return lax.bitcast_convert_type(r, jnp.float32)


def _newton_rsqrt(ss):
    ssi = lax.bitcast_convert_type(ss, jnp.int32)
    yi = jnp.int32(0x5F3759DF) - (ssi >> 1)
    y = lax.bitcast_convert_type(yi, jnp.float32)
    for _ in range(4):
        y = y * (1.5 - 0.5 * ss * y * y)
    return y * (1.0 - 1e-8 * y)


def _simsel_body(lf1d_hbm, lf_hbm, pbs_hbm, tf_hbm, bf_hbm, ti_hbm, bi_hbm,
                 xb0, xb1, pb_v, pbb_v, ti_v, bi_v, gt_v, gb_v, tr_v, br_v,
                 sem_a, sem_b, sem0, sem1):
    wid = lax.axis_index("s") * 2 + lax.axis_index("c")
    base_iota = lax.iota(jnp.int32, L)

    @pl.when(wid < NSC)
    def _():
        bg = SPLIT + wid
        row0 = bg * P
        pltpu.sync_copy(pbs_hbm.at[bg], pb_v)
        pltpu.async_copy(lf1d_hbm.at[pl.ds(row0 * D, L * D)], xb0, sem_a)
        row_off = base_iota * jnp.int32(D)

        def build_pbb(i, _):
            pbv = pb_v[pl.ds(i * L, L)]
            for u in range(L):
                pbb_v[pl.ds((i * L + u) * L, L)] = jnp.full(
                    (L,), pbv[u], jnp.float32)
            return 0
        lax.fori_loop(0, D // L, build_pbb, 0)

        def group_sims(xbuf):
            def p1(i, ss):
                for u in range(L):
                    v = plsc.load_gather(xbuf, [row_off + (i * L + u)])
                    ss = ss + v * v
                return ss
            ss = lax.fori_loop(0, D // L, p1, jnp.zeros((L,), jnp.float32))
            rsq = _newton_rsqrt(ss)

            def p2(i, dot):
                for u in range(L):
                    d = i * L + u
                    v = plsc.load_gather(xbuf, [row_off + d])
                    xn = _round_bf16(v * rsq)
                    dot = dot + xn * pbb_v[pl.ds(d * L, L)]
                return dot
            return lax.fori_loop(0, D // L, p2, jnp.zeros((L,), jnp.float32))

        def super_step(s, carry):
            tk, tv, bk, bv = carry
            gA = 2 * s
            gB = 2 * s + 1
            cp_b = pltpu.async_copy(
                lf1d_hbm.at[pl.ds((row0 + gB * L) * D, L * D)], xb1, sem_b)
            pltpu.make_async_copy(
                lf1d_hbm.at[pl.ds((row0 + gA * L) * D, L * D)], xb0,
                sem_a).wait()
            simA = group_sims(xb0)
            skA, svA = plsc.sort_key_val(simA, base_iota + gA * L)
            tk, tv = _merge_top(tk, tv, skA, svA)
            bk, bv = _merge_bot(bk, bv, skA, svA)

            @pl.when(s < NGRP // 2 - 1)
            def _():
                pltpu.async_copy(
                    lf1d_hbm.at[pl.ds((row0 + (gA + 2) * L) * D, L * D)],
                    xb0, sem_a)
            cp_b.wait()
            simB = group_sims(xb1)
            skB, svB = plsc.sort_key_val(simB, base_iota + gB * L)
            tk, tv = _merge_top(tk, tv, skB, svB)
            bk, bv = _merge_bot(bk, bv, skB, svB)
            return tk, tv, bk, bv

        _, tv, _, bv = lax.fori_loop(0, NGRP // 2, super_step, _SEL_INIT())
        _emit_selection(tv, bv, row0, lf_hbm,
                        tf_hbm.at[wid], bf_hbm.at[wid],
                        ti_hbm.at[wid], bi_hbm.at[wid],
                        ti_v, bi_v, gt_v, gb_v, tr_v, br_v, sem0, sem1)


def _simsel(lf_1d, lf_flat, pbs):
    mesh = plsc.VectorSubcoreMesh(core_axis_name="c", subcore_axis_name="s")
    out_type = (
        jax.ShapeDtypeStruct((NSC, K, D), jnp.float32),
        jax.ShapeDtypeStruct((NSC, K, D), jnp.float32),
        jax.ShapeDtypeStruct((NSC, K), jnp.int32),
        jax.ShapeDtypeStruct((NSC, K), jnp.int32),
    )
    scratch = [
        pltpu.VMEM((L * D,), jnp.float32),
        pltpu.VMEM((L * D,), jnp.float32),
        pltpu.VMEM((D,), jnp.float32),
        pltpu.VMEM((D * L,), jnp.float32),
        pltpu.VMEM((K,), jnp.int32),
        pltpu.VMEM((K,), jnp.int32),
        pltpu.VMEM((K,), jnp.int32),
        pltpu.VMEM((K,), jnp.int32),
        pltpu.VMEM((K, D), jnp.float32),
        pltpu.VMEM((K, D), jnp.float32),
        pltpu.SemaphoreType.DMA,
        pltpu.SemaphoreType.DMA,
        pltpu.SemaphoreType.DMA,
        pltpu.SemaphoreType.DMA,
    ]
    run = pl.kernel(_simsel_body, out_type=out_type, mesh=mesh,
                    scratch_types=scratch,
                    compiler_params=pltpu.CompilerParams(
                        needs_layout_passes=False))
    return run(lf_1d, lf_flat, pbs)


def kernel(local_features, prototypes, labels):
    pbs3 = _build_pbs(prototypes, labels)
    lf_flat = local_features.reshape(B * P, D)
    sc_out = _simsel(local_features.reshape(B * P * D), lf_flat,
                     pbs3.reshape(B, D))
    sim = _similarity(local_features, pbs3)
    tc_out = _select(sim, lf_flat)
    top_feat, bot_feat, top_idx, bot_idx = (
        jnp.concatenate([a, b], axis=0) for a, b in zip(tc_out, sc_out))
    return (top_feat, bot_feat, top_idx, bot_idx)


# speedup vs baseline: 2.0043x; 2.0043x over previous
import functools

import jax
import jax.numpy as jnp
from jax import lax
from jax.experimental import pallas as pl
from jax.experimental.pallas import tpu as pltpu
from jax.experimental.pallas import tpu_sc as plsc

B, P, C, D = 64, 1024, 14, 768
K = 16
L = 16
NW = 32
NCHUNK = P // L
SPLIT = 32
NSC = B - SPLIT
NGRP = P // L


def _pb_body(lbl_ref, proto_ref, out_ref):
    p = proto_ref[0, 0, :]
    pn = p / (jnp.sqrt(jnp.sum(p * p)) + 1e-8)
    out_ref[0, 0, :] = pn.astype(jnp.bfloat16).astype(jnp.float32)


def _build_pbs(prototypes, labels):
    grid_spec = pltpu.PrefetchScalarGridSpec(
        num_scalar_prefetch=1,
        grid=(B,),
        in_specs=[
            pl.BlockSpec((1, 1, D), lambda b, lbl: (b * C + lbl[b], 0, 0)),
        ],
        out_specs=pl.BlockSpec((1, 1, D), lambda b, lbl: (b, 0, 0)),
    )
    pbs3 = pl.pallas_call(
        _pb_body,
        grid_spec=grid_spec,
        out_shape=jax.ShapeDtypeStruct((B, 1, D), jnp.float32),
    )(labels, prototypes.reshape(B * C, 1, D))
    return pbs3


def _sim_body(lf_ref, pb_ref, out_ref):
    pb = pb_ref[0, 0, :]
    x = lf_ref[0]
    nrm = jnp.sqrt(jnp.sum(x * x, axis=1)) + 1e-8
    xb = (x / nrm[:, None]).astype(jnp.bfloat16).astype(jnp.float32)
    out_ref[0, 0, :] = jnp.sum(xb * pb[None, :], axis=1)


def _similarity(local_features, pbs):
    sim3 = pl.pallas_call(
        _sim_body,
        grid=(SPLIT,),
        in_specs=[
            pl.BlockSpec((1, P, D), lambda b: (b, 0, 0)),
            pl.BlockSpec((1, 1, D), lambda b: (b, 0, 0)),
        ],
        out_specs=pl.BlockSpec((1, 1, P), lambda b: (b, 0, 0)),
        out_shape=jax.ShapeDtypeStruct((SPLIT, 1, P), jnp.float32),
    )(local_features, pbs)
    return sim3.reshape(SPLIT, P)


def _merge_top(run_k, run_v, cand_k, cand_v):
    rb_k = lax.rev(cand_k, (0,))
    rb_v = lax.rev(cand_v, (0,))
    keep = run_k >= rb_k
    mk = jnp.where(keep, run_k, rb_k)
    mv = jnp.where(keep, run_v, rb_v)
    return plsc.sort_key_val(mk, mv)


def _merge_bot(run_k, run_v, cand_k, cand_v):
    rb_k = lax.rev(cand_k, (0,))
    rb_v = lax.rev(cand_v, (0,))
    keep = run_k <= rb_k
    mk = jnp.where(keep, run_k, rb_k)
    mv = jnp.where(keep, run_v, rb_v)
    return plsc.sort_key_val(mk, mv)


_SEL_INIT = lambda: (
    jnp.full((L,), -2.0, jnp.float32), jnp.zeros((L,), jnp.int32),
    jnp.full((L,), 2.0, jnp.float32), jnp.zeros((L,), jnp.int32),
)


def _emit_selection(tv, bv, row0, lf_hbm, tf_slot, bf_slot, ti_slot, bi_slot,
                    ti_v, bi_v, gt_v, gb_v, tr_v, br_v, sem0, sem1):
    top_idx = lax.rev(tv, (0,))
    bot_idx = bv
    ti_v[...] = top_idx
    bi_v[...] = bot_idx
    gt_v[...] = top_idx + row0
    gb_v[...] = bot_idx + row0
    cp_t = pltpu.async_copy(lf_hbm.at[gt_v], tr_v, sem0)
    cp_b = pltpu.async_copy(lf_hbm.at[gb_v], br_v, sem1)
    pltpu.sync_copy(ti_v, ti_slot)
    pltpu.sync_copy(bi_v, bi_slot)
    cp_t.wait()
    cp_b.wait()
    pltpu.sync_copy(tr_v, tf_slot)
    pltpu.sync_copy(br_v, bf_slot)


def _select_body(sim_hbm, lf_hbm, tf_hbm, bf_hbm, ti_hbm, bi_hbm,
                 sim_v, ti_v, bi_v, gt_v, gb_v, tr_v, br_v, sem0, sem1):
    wid = lax.axis_index("s") * 2 + lax.axis_index("c")
    base_iota = lax.iota(jnp.int32, L)

    @pl.when(wid < SPLIT)
    def _():
        b = wid
        pltpu.sync_copy(sim_hbm.at[b], sim_v)

        def chunk_step(c, carry):
            tk, tv, bk, bv = carry
            chunk = sim_v[pl.ds(c * L, L)]
            cidx = base_iota + c * L
            sk, sv = plsc.sort_key_val(chunk, cidx)
            tk, tv = _merge_top(tk, tv, sk, sv)
            bk, bv = _merge_bot(bk, bv, sk, sv)
            return tk, tv, bk, bv

        _, tv, _, bv = lax.fori_loop(0, NCHUNK, chunk_step, _SEL_INIT())
        _emit_selection(tv, bv, b * P, lf_hbm,
                        tf_hbm.at[b], bf_hbm.at[b], ti_hbm.at[b], bi_hbm.at[b],
                        ti_v, bi_v, gt_v, gb_v, tr_v, br_v, sem0, sem1)


def _select(sim, lf_flat):
    mesh = plsc.VectorSubcoreMesh(core_axis_name="c", subcore_axis_name="s")
    out_type = (
        jax.ShapeDtypeStruct((SPLIT, K, D), jnp.float32),
        jax.ShapeDtypeStruct((SPLIT, K, D), jnp.float32),
        jax.ShapeDtypeStruct((SPLIT, K), jnp.int32),
        jax.ShapeDtypeStruct((SPLIT, K), jnp.int32),
    )
    scratch = [
        pltpu.VMEM((P,), jnp.float32),
        pltpu.VMEM((K,), jnp.int32),
        pltpu.VMEM((K,), jnp.int32),
        pltpu.VMEM((K,), jnp.int32),
        pltpu.VMEM((K,), jnp.int32),
        pltpu.VMEM((K, D), jnp.float32),
        pltpu.VMEM((K, D), jnp.float32),
        pltpu.SemaphoreType.DMA,
        pltpu.SemaphoreType.DMA,
    ]
    run = pl.kernel(_select_body, out_type=out_type, mesh=mesh,
                    scratch_types=scratch,
                    compiler_params=pltpu.CompilerParams(
                        needs_layout_passes=False))
    return run(sim, lf_flat)


def _round_bf16(x):
    i = lax.bitcast_convert_type(x, jnp.int32)
    r = (i + jnp.int32(0x7FFF) + ((i >> 16) & 1)) & jnp.int32(-65536)
    return lax.bitcast_convert_type(r, jnp.float32)


def _newton_rsqrt(ss):
    ssi = lax.bitcast_convert_type(ss, jnp.int32)
    yi = jnp.int32(0x5F3759DF) - (ssi >> 1)
    y = lax.bitcast_convert_type(yi, jnp.float32)
    for _ in range(4):
        y = y * (1.5 - 0.5 * ss * y * y)
    return y * (1.0 - 1e-8 * y)


def _simsel_body(lf1d_hbm, lf_hbm, pbs_hbm, tf_hbm, bf_hbm, ti_hbm, bi_hbm,
                 xb0, xb1, pb_v, ti_v, bi_v, gt_v, gb_v, tr_v, br_v,
                 sem_a, sem_b, sem0, sem1):
    wid = lax.axis_index("s") * 2 + lax.axis_index("c")
    base_iota = lax.iota(jnp.int32, L)

    @pl.when(wid < NSC)
    def _():
        bg = SPLIT + wid
        row0 = bg * P
        pltpu.sync_copy(pbs_hbm.at[bg], pb_v)
        pltpu.async_copy(lf1d_hbm.at[pl.ds(row0 * D, L * D)], xb0, sem_a)

        def group_sims(xbuf):
            def row_step(r, sim):
                base = r * D
                vs = [xbuf[pl.ds(base + j * L, L)] for j in range(D // L)]
                acc = vs[0] * vs[0]
                for j in range(1, D // L):
                    acc = acc + vs[j] * vs[j]
                ss = jnp.sum(acc)
                rsq = _newton_rsqrt(jnp.full((L,), ss, jnp.float32))
                dot = _round_bf16(vs[0] * rsq) * pb_v[pl.ds(0, L)]
                for j in range(1, D // L):
                    xn = _round_bf16(vs[j] * rsq)
                    dot = dot + xn * pb_v[pl.ds(j * L, L)]
                dots = jnp.sum(dot)
                return jnp.where(base_iota == r, dots, sim)
            return lax.fori_loop(0, L, row_step, jnp.zeros((L,), jnp.float32))

        def super_step(s, carry):
            tk, tv, bk, bv = carry
            gA = 2 * s
            gB = 2 * s + 1
            cp_b = pltpu.async_copy(
                lf1d_hbm.at[pl.ds((row0 + gB * L) * D, L * D)], xb1, sem_b)
            pltpu.make_async_copy(
                lf1d_hbm.at[pl.ds((row0 + gA * L) * D, L * D)], xb0,
                sem_a).wait()
            simA = group_sims(xb0)
            skA, svA = plsc.sort_key_val(simA, base_iota + gA * L)
            tk, tv = _merge_top(tk, tv, skA, svA)
            bk, bv = _merge_bot(bk, bv, skA, svA)

            @pl.when(s < NGRP // 2 - 1)
            def _():
                pltpu.async_copy(
                    lf1d_hbm.at[pl.ds((row0 + (gA + 2) * L) * D, L * D)],
                    xb0, sem_a)
            cp_b.wait()
            simB = group_sims(xb1)
            skB, svB = plsc.sort_key_val(simB, base_iota + gB * L)
            tk, tv = _merge_top(tk, tv, skB, svB)
            bk, bv = _merge_bot(bk, bv, skB, svB)
            return tk, tv, bk, bv

        _, tv, _, bv = lax.fori_loop(0, NGRP // 2, super_step, _SEL_INIT())
        _emit_selection(tv, bv, row0, lf_hbm,
                        tf_hbm.at[wid], bf_hbm.at[wid],
                        ti_hbm.at[wid], bi_hbm.at[wid],
                        ti_v, bi_v, gt_v, gb_v, tr_v, br_v, sem0, sem1)


def _simsel(lf_1d, lf_flat, pbs):
    mesh = plsc.VectorSubcoreMesh(core_axis_name="c", subcore_axis_name="s")
    out_type = (
        jax.ShapeDtypeStruct((NSC, K, D), jnp.float32),
        jax.ShapeDtypeStruct((NSC, K, D), jnp.float32),
        jax.ShapeDtypeStruct((NSC, K), jnp.int32),
        jax.ShapeDtypeStruct((NSC, K), jnp.int32),
    )
    scratch = [
        pltpu.VMEM((L * D,), jnp.float32),
        pltpu.VMEM((L * D,), jnp.float32),
        pltpu.VMEM((D,), jnp.float32),
        pltpu.VMEM((K,), jnp.int32),
        pltpu.VMEM((K,), jnp.int32),
        pltpu.VMEM((K,), jnp.int32),
        pltpu.VMEM((K,), jnp.int32),
        pltpu.VMEM((K, D), jnp.float32),
        pltpu.VMEM((K, D), jnp.float32),
        pltpu.SemaphoreType.DMA,
        pltpu.SemaphoreType.DMA,
        pltpu.SemaphoreType.DMA,
        pltpu.SemaphoreType.DMA,
    ]
    run = pl.kernel(_simsel_body, out_type=out_type, mesh=mesh,
                    scratch_types=scratch,
                    compiler_params=pltpu.CompilerParams(
                        needs_layout_passes=False))
    return run(lf_1d, lf_flat, pbs)


def kernel(local_features, prototypes, labels):
    pbs3 = _build_pbs(prototypes, labels)
    lf_flat = local_features.reshape(B * P, D)
    sc_out = _simsel(local_features.reshape(B * P * D), lf_flat,
                     pbs3.reshape(B, D))
    sim = _similarity(local_features, pbs3)
    tc_out = _select(sim, lf_flat)
    top_feat, bot_feat, top_idx, bot_idx = (
        jnp.concatenate([a, b], axis=0) for a, b in zip(tc_out, sc_out))
    return (top_feat, bot_feat, top_idx, bot_idx)


# speedup vs baseline: 2.2658x; 1.1305x over previous
import functools

import jax
import jax.numpy as jnp
from jax import lax
from jax.experimental import pallas as pl
from jax.experimental.pallas import tpu as pltpu
from jax.experimental.pallas import tpu_sc as plsc

B, P, C, D = 64, 1024, 14, 768
K = 16
L = 16
NW = 32
NCHUNK = P // L
SPLIT = 32
NSC = B - SPLIT
NGRP = P // L


def _pb_body(lbl_ref, proto_ref, out_ref):
    p = proto_ref[0, 0, :]
    pn = p / (jnp.sqrt(jnp.sum(p * p)) + 1e-8)
    out_ref[0, 0, :] = pn.astype(jnp.bfloat16).astype(jnp.float32)


def _build_pbs(prototypes, labels):
    grid_spec = pltpu.PrefetchScalarGridSpec(
        num_scalar_prefetch=1,
        grid=(B,),
        in_specs=[
            pl.BlockSpec((1, 1, D), lambda b, lbl: (b * C + lbl[b], 0, 0)),
        ],
        out_specs=pl.BlockSpec((1, 1, D), lambda b, lbl: (b, 0, 0)),
    )
    pbs3 = pl.pallas_call(
        _pb_body,
        grid_spec=grid_spec,
        out_shape=jax.ShapeDtypeStruct((B, 1, D), jnp.float32),
    )(labels, prototypes.reshape(B * C, 1, D))
    return pbs3


def _sim_body(lf_ref, pb_ref, out_ref):
    pb = pb_ref[0, 0, :]
    x = lf_ref[0]
    nrm = jnp.sqrt(jnp.sum(x * x, axis=1)) + 1e-8
    xb = (x / nrm[:, None]).astype(jnp.bfloat16).astype(jnp.float32)
    out_ref[0, 0, :] = jnp.sum(xb * pb[None, :], axis=1)


def _similarity(local_features, pbs):
    sim3 = pl.pallas_call(
        _sim_body,
        grid=(SPLIT,),
        in_specs=[
            pl.BlockSpec((1, P, D), lambda b: (b, 0, 0)),
            pl.BlockSpec((1, 1, D), lambda b: (b, 0, 0)),
        ],
        out_specs=pl.BlockSpec((1, 1, P), lambda b: (b, 0, 0)),
        out_shape=jax.ShapeDtypeStruct((SPLIT, 1, P), jnp.float32),
    )(local_features, pbs)
    return sim3.reshape(SPLIT, P)


def _merge_top(run_k, run_v, cand_k, cand_v):
    rb_k = lax.rev(cand_k, (0,))
    rb_v = lax.rev(cand_v, (0,))
    keep = run_k >= rb_k
    mk = jnp.where(keep, run_k, rb_k)
    mv = jnp.where(keep, run_v, rb_v)
    return plsc.sort_key_val(mk, mv)


def _merge_bot(run_k, run_v, cand_k, cand_v):
    rb_k = lax.rev(cand_k, (0,))
    rb_v = lax.rev(cand_v, (0,))
    keep = run_k <= rb_k
    mk = jnp.where(keep, run_k, rb_k)
    mv = jnp.where(keep, run_v, rb_v)
    return plsc.sort_key_val(mk, mv)


_SEL_INIT = lambda: (
    jnp.full((L,), -2.0, jnp.float32), jnp.zeros((L,), jnp.int32),
    jnp.full((L,), 2.0, jnp.float32), jnp.zeros((L,), jnp.int32),
)


def _emit_selection(tv, bv, row0, lf_hbm, tf_slot, bf_slot, ti_slot, bi_slot,
                    ti_v, bi_v, gt_v, gb_v, tr_v, br_v, sem0, sem1):
    top_idx = lax.rev(tv, (0,))
    bot_idx = bv
    ti_v[...] = top_idx
    bi_v[...] = bot_idx
    gt_v[...] = top_idx + row0
    gb_v[...] = bot_idx + row0
    cp_t = pltpu.async_copy(lf_hbm.at[gt_v], tr_v, sem0)
    cp_b = pltpu.async_copy(lf_hbm.at[gb_v], br_v, sem1)
    pltpu.sync_copy(ti_v, ti_slot)
    pltpu.sync_copy(bi_v, bi_slot)
    cp_t.wait()
    cp_b.wait()
    pltpu.sync_copy(tr_v, tf_slot)
    pltpu.sync_copy(br_v, bf_slot)


def _select_body(sim_hbm, lf_hbm, tf_hbm, bf_hbm, ti_hbm, bi_hbm,
                 sim_v, ti_v, bi_v, gt_v, gb_v, tr_v, br_v, sem0, sem1):
    wid = lax.axis_index("s") * 2 + lax.axis_index("c")
    base_iota = lax.iota(jnp.int32, L)

    @pl.when(wid < SPLIT)
    def _():
        b = wid
        pltpu.sync_copy(sim_hbm.at[b], sim_v)

        def chunk_step(c, carry):
            tk, tv, bk, bv = carry
            chunk = sim_v[pl.ds(c * L, L)]
            cidx = base_iota + c * L
            sk, sv = plsc.sort_key_val(chunk, cidx)
            tk, tv = _merge_top(tk, tv, sk, sv)
            bk, bv = _merge_bot(bk, bv, sk, sv)
            return tk, tv, bk, bv

        _, tv, _, bv = lax.fori_loop(0, NCHUNK, chunk_step, _SEL_INIT())
        _emit_selection(tv, bv, b * P, lf_hbm,
                        tf_hbm.at[b], bf_hbm.at[b], ti_hbm.at[b], bi_hbm.at[b],
                        ti_v, bi_v, gt_v, gb_v, tr_v, br_v, sem0, sem1)


def _select(sim, lf_flat):
    mesh = plsc.VectorSubcoreMesh(core_axis_name="c", subcore_axis_name="s")
    out_type = (
        jax.ShapeDtypeStruct((SPLIT, K, D), jnp.float32),
        jax.ShapeDtypeStruct((SPLIT, K, D), jnp.float32),
        jax.ShapeDtypeStruct((SPLIT, K), jnp.int32),
        jax.ShapeDtypeStruct((SPLIT, K), jnp.int32),
    )
    scratch = [
        pltpu.VMEM((P,), jnp.float32),
        pltpu.VMEM((K,), jnp.int32),
        pltpu.VMEM((K,), jnp.int32),
        pltpu.VMEM((K,), jnp.int32),
        pltpu.VMEM((K,), jnp.int32),
        pltpu.VMEM((K, D), jnp.float32),
        pltpu.VMEM((K, D), jnp.float32),
        pltpu.SemaphoreType.DMA,
        pltpu.SemaphoreType.DMA,
    ]
    run = pl.kernel(_select_body, out_type=out_type, mesh=mesh,
                    scratch_types=scratch,
                    compiler_params=pltpu.CompilerParams(
                        needs_layout_passes=False))
    return run(sim, lf_flat)


def _round_bf16(x):
    i = lax.bitcast_convert_type(x, jnp.int32)
    r = (i + jnp.int32(0x7FFF) + ((i >> 16) & 1)) & jnp.int32(-65536)
    return lax.bitcast_convert_type(r, jnp.float32)


def _newton_rsqrt(ss):
    ssi = lax.bitcast_convert_type(ss, jnp.int32)
    yi = jnp.int32(0x5F3759DF) - (ssi >> 1)
    y = lax.bitcast_convert_type(yi, jnp.float32)
    for _ in range(4):
        y = y * (1.5 - 0.5 * ss * y * y)
    return y * (1.0 - 1e-8 * y)


def _simsel_body(lf1d_hbm, lf_hbm, pbs_hbm, tf_hbm, bf_hbm, ti_hbm, bi_hbm,
                 xb0, xb1, pb_v, ti_v, bi_v, gt_v, gb_v, tr_v, br_v,
                 sem_a, sem_b, sem0, sem1):
    wid = lax.axis_index("s") * 2 + lax.axis_index("c")
    base_iota = lax.iota(jnp.int32, L)

    @pl.when(wid < NSC)
    def _():
        bg = SPLIT + wid
        row0 = bg * P
        pltpu.sync_copy(pbs_hbm.at[bg], pb_v)
        pltpu.async_copy(lf1d_hbm.at[pl.ds(row0 * D, L * D)], xb0, sem_a)

        def group_sims(xbuf):
            NJ = D // L

            def row_step(r, sim):
                base = r * D
                accs = [jnp.zeros((L,), jnp.float32) for _ in range(4)]
                for j in range(NJ):
                    v = xbuf[pl.ds(base + j * L, L)]
                    accs[j % 4] = accs[j % 4] + v * v
                ss = jnp.sum((accs[0] + accs[1]) + (accs[2] + accs[3]))
                rsq = _newton_rsqrt(jnp.full((L,), ss, jnp.float32))
                dots_ = [jnp.zeros((L,), jnp.float32) for _ in range(4)]
                for j in range(NJ):
                    xn = _round_bf16(xbuf[pl.ds(base + j * L, L)] * rsq)
                    dots_[j % 4] = dots_[j % 4] + xn * pb_v[pl.ds(j * L, L)]
                dots = jnp.sum((dots_[0] + dots_[1]) + (dots_[2] + dots_[3]))
                return jnp.where(base_iota == r, dots, sim)
            return lax.fori_loop(0, L, row_step, jnp.zeros((L,), jnp.float32))

        def super_step(s, carry):
            tk, tv, bk, bv = carry
            gA = 2 * s
            gB = 2 * s + 1
            cp_b = pltpu.async_copy(
                lf1d_hbm.at[pl.ds((row0 + gB * L) * D, L * D)], xb1, sem_b)
            pltpu.make_async_copy(
                lf1d_hbm.at[pl.ds((row0 + gA * L) * D, L * D)], xb0,
                sem_a).wait()
            simA = group_sims(xb0)
            skA, svA = plsc.sort_key_val(simA, base_iota + gA * L)
            tk, tv = _merge_top(tk, tv, skA, svA)
            bk, bv = _merge_bot(bk, bv, skA, svA)

            @pl.when(s < NGRP // 2 - 1)
            def _():
                pltpu.async_copy(
                    lf1d_hbm.at[pl.ds((row0 + (gA + 2) * L) * D, L * D)],
                    xb0, sem_a)
            cp_b.wait()
            simB = group_sims(xb1)
            skB, svB = plsc.sort_key_val(simB, base_iota + gB * L)
            tk, tv = _merge_top(tk, tv, skB, svB)
            bk, bv = _merge_bot(bk, bv, skB, svB)
            return tk, tv, bk, bv

        _, tv, _, bv = lax.fori_loop(0, NGRP // 2, super_step, _SEL_INIT())
        _emit_selection(tv, bv, row0, lf_hbm,
                        tf_hbm.at[wid], bf_hbm.at[wid],
                        ti_hbm.at[wid], bi_hbm.at[wid],
                        ti_v, bi_v, gt_v, gb_v, tr_v, br_v, sem0, sem1)


def _simsel(lf_1d, lf_flat, pbs):
    mesh = plsc.VectorSubcoreMesh(core_axis_name="c", subcore_axis_name="s")
    out_type = (
        jax.ShapeDtypeStruct((NSC, K, D), jnp.float32),
        jax.ShapeDtypeStruct((NSC, K, D), jnp.float32),
        jax.ShapeDtypeStruct((NSC, K), jnp.int32),
        jax.ShapeDtypeStruct((NSC, K), jnp.int32),
    )
    scratch = [
        pltpu.VMEM((L * D,), jnp.float32),
        pltpu.VMEM((L * D,), jnp.float32),
        pltpu.VMEM((D,), jnp.float32),
        pltpu.VMEM((K,), jnp.int32),
        pltpu.VMEM((K,), jnp.int32),
        pltpu.VMEM((K,), jnp.int32),
        pltpu.VMEM((K,), jnp.int32),
        pltpu.VMEM((K, D), jnp.float32),
        pltpu.VMEM((K, D), jnp.float32),
        pltpu.SemaphoreType.DMA,
        pltpu.SemaphoreType.DMA,
        pltpu.SemaphoreType.DMA,
        pltpu.SemaphoreType.DMA,
    ]
    run = pl.kernel(_simsel_body, out_type=out_type, mesh=mesh,
                    scratch_types=scratch,
                    compiler_params=pltpu.CompilerParams(
                        needs_layout_passes=False))
    return run(lf_1d, lf_flat, pbs)


def kernel(local_features, prototypes, labels):
    pbs3 = _build_pbs(prototypes, labels)
    lf_flat = local_features.reshape(B * P, D)
    sc_out = _simsel(local_features.reshape(B * P * D), lf_flat,
                     pbs3.reshape(B, D))
    sim = _similarity(local_features, pbs3)
    tc_out = _select(sim, lf_flat)
    top_feat, bot_feat, top_idx, bot_idx = (
        jnp.concatenate([a, b], axis=0) for a, b in zip(tc_out, sc_out))
    return (top_feat, bot_feat, top_idx, bot_idx)


# speedup vs baseline: 5.7302x; 2.5290x over previous
import functools

import jax
import jax.numpy as jnp
from jax import lax
from jax.experimental import pallas as pl
from jax.experimental.pallas import tpu as pltpu
from jax.experimental.pallas import tpu_sc as plsc

B, P, C, D = 64, 1024, 14, 768
K = 16
L = 16
NW = 32
BATCHES_PER_W = B // NW
NCHUNK = P // L


def _pb_body(lbl_ref, proto_ref, out_ref):
    p = proto_ref[0, 0, :]
    pn = p / (jnp.sqrt(jnp.sum(p * p)) + 1e-8)
    out_ref[0, 0, :] = pn.astype(jnp.bfloat16).astype(jnp.float32)


def _build_pbs(prototypes, labels):
    grid_spec = pltpu.PrefetchScalarGridSpec(
        num_scalar_prefetch=1,
        grid=(B,),
        in_specs=[
            pl.BlockSpec((1, 1, D), lambda b, lbl: (b * C + lbl[b], 0, 0)),
        ],
        out_specs=pl.BlockSpec((1, 1, D), lambda b, lbl: (b, 0, 0)),
    )
    return pl.pallas_call(
        _pb_body,
        grid_spec=grid_spec,
        out_shape=jax.ShapeDtypeStruct((B, 1, D), jnp.float32),
    )(labels, prototypes.reshape(B * C, 1, D))


def _sim_body(lf_ref, pb_ref, out_ref):
    pb = pb_ref[0, 0, :]
    x = lf_ref[0]
    nrm = jnp.sqrt(jnp.sum(x * x, axis=1)) + 1e-8
    xb = (x / nrm[:, None]).astype(jnp.bfloat16).astype(jnp.float32)
    out_ref[0, 0, :] = jnp.sum(xb * pb[None, :], axis=1)


def _similarity(local_features, pbs):
    sim3 = pl.pallas_call(
        _sim_body,
        grid=(B,),
        in_specs=[
            pl.BlockSpec((1, P, D), lambda b: (b, 0, 0)),
            pl.BlockSpec((1, 1, D), lambda b: (b, 0, 0)),
        ],
        out_specs=pl.BlockSpec((1, 1, P), lambda b: (b, 0, 0)),
        out_shape=jax.ShapeDtypeStruct((B, 1, P), jnp.float32),
    )(local_features, pbs)
    return sim3.reshape(B, P)


def _merge_top(run_k, run_v, cand_k, cand_v):
    rb_k = lax.rev(cand_k, (0,))
    rb_v = lax.rev(cand_v, (0,))
    keep = run_k >= rb_k
    mk = jnp.where(keep, run_k, rb_k)
    mv = jnp.where(keep, run_v, rb_v)
    return plsc.sort_key_val(mk, mv)


def _merge_bot(run_k, run_v, cand_k, cand_v):
    rb_k = lax.rev(cand_k, (0,))
    rb_v = lax.rev(cand_v, (0,))
    keep = run_k <= rb_k
    mk = jnp.where(keep, run_k, rb_k)
    mv = jnp.where(keep, run_v, rb_v)
    return plsc.sort_key_val(mk, mv)


def _select_body(sim_hbm, lf_hbm, tf_hbm, bf_hbm, ti_hbm, bi_hbm,
                 sim_v, ti_v, bi_v, gt_v, gb_v, tr_v, br_v, sem0, sem1):
    wid = lax.axis_index("s") * 2 + lax.axis_index("c")
    base_iota = lax.iota(jnp.int32, L)
    for j in range(BATCHES_PER_W):
        b = wid * BATCHES_PER_W + j
        pltpu.sync_copy(sim_hbm.at[b], sim_v)

        def chunk_step(c, carry):
            tk, tv, bk, bv = carry
            chunk = sim_v[pl.ds(c * L, L)]
            cidx = base_iota + c * L
            sk, sv = plsc.sort_key_val(chunk, cidx)
            tk, tv = _merge_top(tk, tv, sk, sv)
            bk, bv = _merge_bot(bk, bv, sk, sv)
            return tk, tv, bk, bv

        init = (
            jnp.full((L,), -2.0, jnp.float32), jnp.zeros((L,), jnp.int32),
            jnp.full((L,), 2.0, jnp.float32), jnp.zeros((L,), jnp.int32),
        )
        _, tv, _, bv = lax.fori_loop(0, NCHUNK, chunk_step, init)

        top_idx = lax.rev(tv, (0,))
        bot_idx = bv
        ti_v[...] = top_idx
        bi_v[...] = bot_idx
        gt_v[...] = top_idx + b * P
        gb_v[...] = bot_idx + b * P
        cp_t = pltpu.async_copy(lf_hbm.at[gt_v], tr_v, sem0)
        cp_b = pltpu.async_copy(lf_hbm.at[gb_v], br_v, sem1)
        pltpu.sync_copy(ti_v, ti_hbm.at[b])
        pltpu.sync_copy(bi_v, bi_hbm.at[b])
        cp_t.wait()
        cp_b.wait()
        pltpu.sync_copy(tr_v, tf_hbm.at[b])
        pltpu.sync_copy(br_v, bf_hbm.at[b])


def _select(sim, lf_flat):
    mesh = plsc.VectorSubcoreMesh(core_axis_name="c", subcore_axis_name="s")
    out_type = (
        jax.ShapeDtypeStruct((B, K, D), jnp.float32),
        jax.ShapeDtypeStruct((B, K, D), jnp.float32),
        jax.ShapeDtypeStruct((B, K), jnp.int32),
        jax.ShapeDtypeStruct((B, K), jnp.int32),
    )
    scratch = [
        pltpu.VMEM((P,), jnp.float32),
        pltpu.VMEM((K,), jnp.int32),
        pltpu.VMEM((K,), jnp.int32),
        pltpu.VMEM((K,), jnp.int32),
        pltpu.VMEM((K,), jnp.int32),
        pltpu.VMEM((K, D), jnp.float32),
        pltpu.VMEM((K, D), jnp.float32),
        pltpu.SemaphoreType.DMA,
        pltpu.SemaphoreType.DMA,
    ]
    run = pl.kernel(_select_body, out_type=out_type, mesh=mesh,
                    scratch_types=scratch,
                    compiler_params=pltpu.CompilerParams(
                        needs_layout_passes=False))
    return run(sim, lf_flat)


def kernel(local_features, prototypes, labels):
    pbs3 = _build_pbs(prototypes, labels)
    sim = _similarity(local_features, pbs3)
    lf_flat = local_features.reshape(B * P, D)
    top_feat, bot_feat, top_idx, bot_idx = _select(sim, lf_flat)
    return (top_feat, bot_feat, top_idx, bot_idx)


# speedup vs baseline: 6.9814x; 1.2183x over previous
import functools

import jax
import jax.numpy as jnp
from jax import lax
from jax.experimental import pallas as pl
from jax.experimental.pallas import tpu as pltpu
from jax.experimental.pallas import tpu_sc as plsc

B, P, C, D = 64, 1024, 14, 768
K = 16
L = 16
NW = 32
BATCHES_PER_W = B // NW
NCHUNK = P // L


def _sim_body(lbl_ref, lf_ref, proto_ref, out_ref):
    p = proto_ref[0, 0, :]
    pn = p / (jnp.sqrt(jnp.sum(p * p)) + 1e-8)
    pb = pn.astype(jnp.bfloat16).astype(jnp.float32)
    x = lf_ref[0]
    nrm = jnp.sqrt(jnp.sum(x * x, axis=1)) + 1e-8
    xb = (x / nrm[:, None]).astype(jnp.bfloat16).astype(jnp.float32)
    out_ref[0, 0, :] = jnp.sum(xb * pb[None, :], axis=1)


def _similarity(local_features, prototypes, labels):
    grid_spec = pltpu.PrefetchScalarGridSpec(
        num_scalar_prefetch=1,
        grid=(B,),
        in_specs=[
            pl.BlockSpec((1, P, D), lambda b, lbl: (b, 0, 0)),
            pl.BlockSpec((1, 1, D), lambda b, lbl: (b * C + lbl[b], 0, 0)),
        ],
        out_specs=pl.BlockSpec((1, 1, P), lambda b, lbl: (b, 0, 0)),
    )
    proto_flat = prototypes.reshape(B * C, 1, D)
    sim3 = pl.pallas_call(
        _sim_body,
        grid_spec=grid_spec,
        out_shape=jax.ShapeDtypeStruct((B, 1, P), jnp.float32),
    )(labels, local_features, proto_flat)
    return sim3.reshape(B, P)


def _merge_top(run_k, run_v, cand_k, cand_v):
    rb_k = lax.rev(cand_k, (0,))
    rb_v = lax.rev(cand_v, (0,))
    keep = run_k >= rb_k
    mk = jnp.where(keep, run_k, rb_k)
    mv = jnp.where(keep, run_v, rb_v)
    return plsc.sort_key_val(mk, mv)


def _merge_bot(run_k, run_v, cand_k, cand_v):
    rb_k = lax.rev(cand_k, (0,))
    rb_v = lax.rev(cand_v, (0,))
    keep = run_k <= rb_k
    mk = jnp.where(keep, run_k, rb_k)
    mv = jnp.where(keep, run_v, rb_v)
    return plsc.sort_key_val(mk, mv)


def _select_body(sim_hbm, lf_hbm, tf_hbm, bf_hbm, ti_hbm, bi_hbm,
                 sim_v, ti_v, bi_v, gt_v, gb_v, tr_v, br_v, sem0, sem1):
    wid = lax.axis_index("s") * 2 + lax.axis_index("c")
    base_iota = lax.iota(jnp.int32, L)
    for j in range(BATCHES_PER_W):
        b = wid * BATCHES_PER_W + j
        pltpu.sync_copy(sim_hbm.at[b], sim_v)

        def chunk_step(c, carry):
            tk, tv, bk, bv = carry
            chunk = sim_v[pl.ds(c * L, L)]
            cidx = base_iota + c * L
            sk, sv = plsc.sort_key_val(chunk, cidx)
            tk, tv = _merge_top(tk, tv, sk, sv)
            bk, bv = _merge_bot(bk, bv, sk, sv)
            return tk, tv, bk, bv

        init = (
            jnp.full((L,), -2.0, jnp.float32), jnp.zeros((L,), jnp.int32),
            jnp.full((L,), 2.0, jnp.float32), jnp.zeros((L,), jnp.int32),
        )
        _, tv, _, bv = lax.fori_loop(0, NCHUNK, chunk_step, init)

        top_idx = lax.rev(tv, (0,))
        bot_idx = bv
        ti_v[...] = top_idx
        bi_v[...] = bot_idx
        gt_v[...] = top_idx + b * P
        gb_v[...] = bot_idx + b * P
        cp_t = pltpu.async_copy(lf_hbm.at[gt_v], tr_v, sem0)
        cp_b = pltpu.async_copy(lf_hbm.at[gb_v], br_v, sem1)
        pltpu.sync_copy(ti_v, ti_hbm.at[b])
        pltpu.sync_copy(bi_v, bi_hbm.at[b])
        cp_t.wait()
        cp_b.wait()
        pltpu.sync_copy(tr_v, tf_hbm.at[b])
        pltpu.sync_copy(br_v, bf_hbm.at[b])


def _select(sim, lf_flat):
    mesh = plsc.VectorSubcoreMesh(core_axis_name="c", subcore_axis_name="s")
    out_type = (
        jax.ShapeDtypeStruct((B, K, D), jnp.float32),
        jax.ShapeDtypeStruct((B, K, D), jnp.float32),
        jax.ShapeDtypeStruct((B, K), jnp.int32),
        jax.ShapeDtypeStruct((B, K), jnp.int32),
    )
    scratch = [
        pltpu.VMEM((P,), jnp.float32),
        pltpu.VMEM((K,), jnp.int32),
        pltpu.VMEM((K,), jnp.int32),
        pltpu.VMEM((K,), jnp.int32),
        pltpu.VMEM((K,), jnp.int32),
        pltpu.VMEM((K, D), jnp.float32),
        pltpu.VMEM((K, D), jnp.float32),
        pltpu.SemaphoreType.DMA,
        pltpu.SemaphoreType.DMA,
    ]
    run = pl.kernel(_select_body, out_type=out_type, mesh=mesh,
                    scratch_types=scratch,
                    compiler_params=pltpu.CompilerParams(
                        needs_layout_passes=False))
    return run(sim, lf_flat)


def kernel(local_features, prototypes, labels):
    sim = _similarity(local_features, prototypes, labels)
    lf_flat = local_features.reshape(B * P, D)
    top_feat, bot_feat, top_idx, bot_idx = _select(sim, lf_flat)
    return (top_feat, bot_feat, top_idx, bot_idx)
